# Initial kernel scaffold; baseline (speedup 1.0000x reference)
import functools

import jax
import jax.numpy as jnp
from jax import lax
from jax.experimental import pallas as pl
from jax.experimental.pallas import tpu as pltpu
from jax.experimental.pallas import tpu_sc as plsc

_SIGMA = 0.5
_L = 16


def _y_matmul_body(x_ref, w_ref, o_ref):
    o_ref[0] = jnp.dot(x_ref[...], w_ref[0], preferred_element_type=jnp.float32)


def _rsqrt(d2):
    i = plsc.bitcast(d2, jnp.int32)
    i = jnp.int32(0x5F3759DF) - (i >> 1)
    y = plsc.bitcast(i, jnp.float32)
    for _ in range(3):
        y = y * (1.5 - 0.5 * d2 * y * y)
    return y


@functools.lru_cache(maxsize=None)
def _make_sc_kernel(N, E, K, D):
    NC, NS = 2, 16
    NW = NC * NS
    EPW = E // NW
    GPW = EPW // _L
    RPT = N // NS
    ZR = 25
    NPAIR = (GPW + 1) // 2
    R8 = D // _L

    mesh = plsc.VectorSubcoreMesh(core_axis_name="c", subcore_axis_name="s")

    @functools.partial(
        pl.kernel,
        out_type=jax.ShapeDtypeStruct((NC, N, D), jnp.float32),
        mesh=mesh,
        scratch_types=[
            pltpu.VMEM((3, N), jnp.float32),
            pltpu.VMEM((3, _L), jnp.float32),
            pltpu.VMEM((EPW,), jnp.int32),
            pltpu.VMEM((EPW,), jnp.int32),
            pltpu.VMEM((2, K, _L, D), jnp.float32),
            pltpu.VMEM((2, K, _L), jnp.float32),
            pltpu.VMEM((2, _L, D), jnp.float32),
            pltpu.VMEM((ZR, D), jnp.float32),
            pltpu.VMEM_SHARED((N, D), jnp.float32),
            pltpu.SemaphoreType.DMA,
            pltpu.SemaphoreType.DMA,
            pltpu.SemaphoreType.DMA,
            pltpu.SemaphoreType.DMA,
        ],
    )
    def sc(y_hbm, src_hbm, dst_hbm, pos_hbm, kp_hbm, out_hbm,
           posv, kpv, srcv, dstv, rowv, inflv, msgv, zerov, outacc,
           gsem0, gsem1, ssem0, ssem1):
        cid = lax.axis_index("c")
        sid = lax.axis_index("s")
        wid = cid * NS + sid

        pltpu.sync_copy(pos_hbm, posv)
        pltpu.sync_copy(kp_hbm, kpv)
        pltpu.sync_copy(src_hbm.at[pl.ds(wid * EPW, EPW)], srcv)
        pltpu.sync_copy(dst_hbm.at[pl.ds(wid * EPW, EPW)], dstv)

        def zb(i, _):
            for r in range(R8):
                zerov[i, pl.ds(r * _L, _L)] = jnp.zeros((_L,), jnp.float32)
            return 0
        lax.fori_loop(0, ZR, zb, 0)
        base = sid * RPT

        def zc(i, _):
            pltpu.sync_copy(zerov, outacc.at[pl.ds(base + i * ZR, ZR)])
            return 0
        lax.fori_loop(0, RPT // ZR, zc, 0)
        plsc.subcore_barrier()

        kps = [(kpv[0, k], kpv[1, k], kpv[2, k]) for k in range(K)]

        def issue_gathers(g, b, gsem):
            sv = srcv[pl.ds(g * _L, _L)]
            for k in range(K):
                pltpu.async_copy(y_hbm.at[sv + k * N], rowv.at[b].at[k], gsem)

        def process(g, b, gsem, ssem):
            sv = srcv[pl.ds(g * _L, _L)]
            dv = dstv[pl.ds(g * _L, _L)]
            pxs = plsc.load_gather(posv.at[0], [sv])
            pys = plsc.load_gather(posv.at[1], [sv])
            pzs = plsc.load_gather(posv.at[2], [sv])
            pxd = plsc.load_gather(posv.at[0], [dv])
            pyd = plsc.load_gather(posv.at[1], [dv])
            pzd = plsc.load_gather(posv.at[2], [dv])
            relx = pxs - pxd
            rely = pys - pyd
            relz = pzs - pzd
            for k in range(K):
                kx, ky, kz = kps[k]
                dx = relx - kx
                dy = rely - ky
                dz = relz - kz
                d2 = dx * dx + dy * dy + dz * dz + 1e-12
                dist = d2 * _rsqrt(d2)
                inflv[b, k, :] = jnp.maximum(0.0, 1.0 - dist * (1.0 / _SIGMA))
            for k in range(K):
                pltpu.make_async_copy(
                    y_hbm.at[pl.ds(0, _L)], rowv.at[b].at[k], gsem).wait()

            @pl.when(g >= 2)
            def _():
                pltpu.make_async_copy(
                    y_hbm.at[pl.ds(0, _L)], msgv.at[b], ssem).wait()

            def jbody(j, _):
                accs = [jnp.zeros((_L,), jnp.float32) for _ in range(R8)]
                for k in range(K):
                    s = inflv[b, k, j]
                    for r in range(R8):
                        accs[r] = accs[r] + rowv[b, k, j, pl.ds(r * _L, _L)] * s
                for r in range(R8):
                    msgv[b, j, pl.ds(r * _L, _L)] = accs[r]
                return 0
            lax.fori_loop(0, _L, jbody, 0)
            pltpu.async_copy(msgv.at[b], outacc.at[dv], ssem, add=True)

        issue_gathers(0, 0, gsem0)

        def pair(gg, _):
            g0 = gg * 2
            g1 = g0 + 1

            @pl.when(g1 < GPW)
            def _():
                issue_gathers(g1, 1, gsem1)
            process(g0, 0, gsem0, ssem0)

            @pl.when(g0 + 2 < GPW)
            def _():
                issue_gathers(g0 + 2, 0, gsem0)

            @pl.when(g1 < GPW)
            def _():
                process(g1, 1, gsem1, ssem1)
            return 0
        lax.fori_loop(0, NPAIR, pair, 0)

        pltpu.make_async_copy(y_hbm.at[pl.ds(0, _L)], msgv.at[0], ssem0).wait()
        pltpu.make_async_copy(y_hbm.at[pl.ds(0, _L)], msgv.at[1], ssem1).wait()
        plsc.subcore_barrier()

        pltpu.sync_copy(outacc.at[pl.ds(base, RPT)],
                        out_hbm.at[cid].at[pl.ds(base, RPT)])

    return sc


def kernel(x, pos, edge_index, kernel_points, weights):
    N, D_IN = x.shape
    K = kernel_points.shape[0]
    D_OUT = weights.shape[2]
    E = edge_index.shape[1]

    y = pl.pallas_call(
        _y_matmul_body,
        grid=(K,),
        in_specs=[
            pl.BlockSpec((N, D_IN), lambda k: (0, 0)),
            pl.BlockSpec((1, D_IN, D_OUT), lambda k: (k, 0, 0)),
        ],
        out_specs=pl.BlockSpec((1, N, D_OUT), lambda k: (k, 0, 0)),
        out_shape=jax.ShapeDtypeStruct((K, N, D_OUT), jnp.float32),
    )(x, weights)

    src = edge_index[0]
    dst = edge_index[1]
    pos_t = pos.T
    kp = jnp.zeros((3, _L), jnp.float32).at[:, :K].set(kernel_points.T)

    sc = _make_sc_kernel(N, E, K, D_OUT)
    partials = sc(y.reshape(K * N, D_OUT), src, dst, pos_t, kp)
    return partials[0] + partials[1]


# speedup vs baseline: 6.1139x; 6.1139x over previous
import functools

import jax
import jax.numpy as jnp
from jax import lax
from jax.experimental import pallas as pl
from jax.experimental.pallas import tpu as pltpu
from jax.experimental.pallas import tpu_sc as plsc

_SIGMA = 0.5
_L = 16


def _y_matmul_body(x_ref, w_ref, o_ref):
    o_ref[0] = jnp.dot(x_ref[...], w_ref[0], preferred_element_type=jnp.float32)


def _rsqrt(d2):
    i = plsc.bitcast(d2, jnp.int32)
    i = jnp.int32(0x5F3759DF) - (i >> 1)
    y = plsc.bitcast(i, jnp.float32)
    for _ in range(3):
        y = y * (1.5 - 0.5 * d2 * y * y)
    return y


@functools.lru_cache(maxsize=None)
def _make_sc_kernel(N, E, K, D):
    NC, NS = 2, 16
    NW = NC * NS
    EPW = E // NW
    GPW = EPW // _L
    RPT = N // NS
    ZR = 25
    R8 = D // _L
    C0 = 8
    C1 = K - C0
    EB = 2000
    GPB = EB // _L
    NBLK = EPW // EB

    mesh = plsc.VectorSubcoreMesh(core_axis_name="c", subcore_axis_name="s")

    @functools.partial(
        pl.kernel,
        out_type=jax.ShapeDtypeStruct((NC, N, D), jnp.float32),
        mesh=mesh,
        compiler_params=pltpu.CompilerParams(
            use_tc_tiling_on_sc=False, needs_layout_passes=False),
        scratch_types=[
            pltpu.VMEM((48,), jnp.float32),
            pltpu.VMEM((EB,), jnp.int32),
            pltpu.VMEM((EB,), jnp.int32),
            pltpu.VMEM((2, C0, _L, D), jnp.float32),
            pltpu.VMEM((2, 6, _L), jnp.float32),
            pltpu.VMEM((2, _L, D), jnp.float32),
            pltpu.VMEM((ZR, D), jnp.float32),
            pltpu.VMEM_SHARED((N, D), jnp.float32),
            pltpu.SemaphoreType.DMA,
            pltpu.SemaphoreType.DMA,
            pltpu.SemaphoreType.DMA,
            pltpu.SemaphoreType.DMA,
            pltpu.SemaphoreType.DMA,
        ],
    )
    def sc(y_hbm, src_hbm, dst_hbm, px_hbm, py_hbm, pz_hbm, kp_hbm, out_hbm,
           kpv, srcv, dstv, rowv, posb, msgv, zerov, outacc,
           gsem0, gsem1, psem, ssem0, ssem1):
        cid = lax.axis_index("c")
        sid = lax.axis_index("s")
        wid = cid * NS + sid
        ebase = wid * EPW

        pltpu.sync_copy(kp_hbm, kpv)

        def zb(i, _):
            for r in range(R8):
                zerov[i, pl.ds(r * _L, _L)] = jnp.zeros((_L,), jnp.float32)
            return 0
        lax.fori_loop(0, ZR, zb, 0)
        base = sid * RPT

        def zc(i, _):
            pltpu.sync_copy(zerov, outacc.at[pl.ds(base + i * ZR, ZR)])
            return 0
        lax.fori_loop(0, RPT // ZR, zc, 0)
        plsc.subcore_barrier()

        kxv = kpv[pl.ds(0, _L)]
        kyv = kpv[pl.ds(_L, _L)]
        kzv = kpv[pl.ds(2 * _L, _L)]
        kps = [(kxv[k], kyv[k], kzv[k]) for k in range(K)]

        def issue_chunk0(gl, b):
            sv = srcv[pl.ds(gl * _L, _L)]
            dv = dstv[pl.ds(gl * _L, _L)]
            for k in range(C0):
                pltpu.async_copy(y_hbm.at[sv + k * N], rowv.at[0].at[k], gsem0)
            pltpu.async_copy(px_hbm.at[sv], posb.at[b].at[0], psem)
            pltpu.async_copy(py_hbm.at[sv], posb.at[b].at[1], psem)
            pltpu.async_copy(pz_hbm.at[sv], posb.at[b].at[2], psem)
            pltpu.async_copy(px_hbm.at[dv], posb.at[b].at[3], psem)
            pltpu.async_copy(py_hbm.at[dv], posb.at[b].at[4], psem)
            pltpu.async_copy(pz_hbm.at[dv], posb.at[b].at[5], psem)

        def body(g, _):
            gl = g % GPB
            b = g % 2

            @pl.when(gl == 0)
            def _():
                blk = g // GPB
                pltpu.sync_copy(src_hbm.at[pl.ds(ebase + blk * EB, EB)], srcv)
                pltpu.sync_copy(dst_hbm.at[pl.ds(ebase + blk * EB, EB)], dstv)
                issue_chunk0(0, b)

            sv = srcv[pl.ds(gl * _L, _L)]
            dv = dstv[pl.ds(gl * _L, _L)]

            for k in range(C1):
                pltpu.async_copy(
                    y_hbm.at[sv + (C0 + k) * N], rowv.at[1].at[k], gsem1)

            for c in range(6):
                pltpu.make_async_copy(
                    px_hbm.at[pl.ds(0, _L)], posb.at[b].at[c], psem).wait()
            relx = posb[b, 0, :] - posb[b, 3, :]
            rely = posb[b, 1, :] - posb[b, 4, :]
            relz = posb[b, 2, :] - posb[b, 5, :]
            infls = []
            for k in range(K):
                kx, ky, kz = kps[k]
                dx = relx - kx
                dy = rely - ky
                dz = relz - kz
                d2 = dx * dx + dy * dy + dz * dz + 1e-12
                dist = d2 * _rsqrt(d2)
                infls.append(jnp.maximum(0.0, 1.0 - dist * (1.0 / _SIGMA)))

            @pl.when(jnp.logical_and(g >= 2, b == 0))
            def _():
                pltpu.make_async_copy(
                    y_hbm.at[pl.ds(0, _L)], msgv.at[0], ssem0).wait()

            @pl.when(jnp.logical_and(g >= 2, b == 1))
            def _():
                pltpu.make_async_copy(
                    y_hbm.at[pl.ds(0, _L)], msgv.at[1], ssem1).wait()

            for k in range(C0):
                pltpu.make_async_copy(
                    y_hbm.at[pl.ds(0, _L)], rowv.at[0].at[k], gsem0).wait()
            for j in range(_L):
                accs = [jnp.zeros((_L,), jnp.float32) for _ in range(R8)]
                for k in range(C0):
                    s = infls[k][j]
                    for r in range(R8):
                        accs[r] = accs[r] + rowv[0, k, j, pl.ds(r * _L, _L)] * s
                for r in range(R8):
                    msgv[b, j, pl.ds(r * _L, _L)] = accs[r]

            @pl.when(jnp.logical_and(g + 1 < GPW, gl + 1 < GPB))
            def _():
                issue_chunk0(gl + 1, 1 - b)

            for k in range(C1):
                pltpu.make_async_copy(
                    y_hbm.at[pl.ds(0, _L)], rowv.at[1].at[k], gsem1).wait()
            for j in range(_L):
                accs = [msgv[b, j, pl.ds(r * _L, _L)] for r in range(R8)]
                for k in range(C1):
                    s = infls[C0 + k][j]
                    for r in range(R8):
                        accs[r] = accs[r] + rowv[1, k, j, pl.ds(r * _L, _L)] * s
                for r in range(R8):
                    msgv[b, j, pl.ds(r * _L, _L)] = accs[r]

            @pl.when(b == 0)
            def _():
                pltpu.async_copy(msgv.at[0], outacc.at[dv], ssem0, add=True)

            @pl.when(b == 1)
            def _():
                pltpu.async_copy(msgv.at[1], outacc.at[dv], ssem1, add=True)
            return 0

        lax.fori_loop(0, GPW, body, 0)

        pltpu.make_async_copy(y_hbm.at[pl.ds(0, _L)], msgv.at[0], ssem0).wait()
        pltpu.make_async_copy(y_hbm.at[pl.ds(0, _L)], msgv.at[1], ssem1).wait()
        plsc.subcore_barrier()

        pltpu.sync_copy(outacc.at[pl.ds(base, RPT)],
                        out_hbm.at[cid].at[pl.ds(base, RPT)])

    return sc


def kernel(x, pos, edge_index, kernel_points, weights):
    N, D_IN = x.shape
    K = kernel_points.shape[0]
    D_OUT = weights.shape[2]
    E = edge_index.shape[1]

    y = pl.pallas_call(
        _y_matmul_body,
        grid=(K,),
        in_specs=[
            pl.BlockSpec((N, D_IN), lambda k: (0, 0)),
            pl.BlockSpec((1, D_IN, D_OUT), lambda k: (k, 0, 0)),
        ],
        out_specs=pl.BlockSpec((1, N, D_OUT), lambda k: (k, 0, 0)),
        out_shape=jax.ShapeDtypeStruct((K, N, D_OUT), jnp.float32),
    )(x, weights)

    src = edge_index[0]
    dst = edge_index[1]
    kp = jnp.zeros((3, _L), jnp.float32).at[:, :K].set(kernel_points.T).reshape(3 * _L)

    sc = _make_sc_kernel(N, E, K, D_OUT)
    partials = sc(y.reshape(K * N, D_OUT), src, dst,
                  pos[:, 0], pos[:, 1], pos[:, 2], kp)
    return partials[0] + partials[1]


# speedup vs baseline: 17.7961x; 2.9108x over previous
import functools

import jax
import jax.numpy as jnp
from jax import lax
from jax.experimental import pallas as pl
from jax.experimental.pallas import tpu as pltpu
from jax.experimental.pallas import tpu_sc as plsc

_SIGMA = 0.5
_L = 16


def _y_matmul_body(x_ref, w_ref, o_ref):
    o_ref[0] = jnp.dot(x_ref[...], w_ref[0], preferred_element_type=jnp.float32)


def _rsqrt(d2):
    i = plsc.bitcast(d2, jnp.int32)
    i = jnp.int32(0x5F3759DF) - (i >> 1)
    y = plsc.bitcast(i, jnp.float32)
    for _ in range(3):
        y = y * (1.5 - 0.5 * d2 * y * y)
    return y


@functools.lru_cache(maxsize=None)
def _make_sc_kernel(N, E, K, D):
    NC, NS = 2, 16
    NW = NC * NS
    EPW = E // NW
    GPW = EPW // _L
    RPT = N // NS
    ZR = 25
    R8 = D // _L
    EB = 2000
    GPB = EB // _L
    CAP = 256
    NSLOT = 4

    mesh = plsc.VectorSubcoreMesh(core_axis_name="c", subcore_axis_name="s")

    @functools.partial(
        pl.kernel,
        out_type=jax.ShapeDtypeStruct((NC, N, D), jnp.float32),
        mesh=mesh,
        compiler_params=pltpu.CompilerParams(
            use_tc_tiling_on_sc=False, needs_layout_passes=False),
        scratch_types=[
            pltpu.VMEM((3 * _L,), jnp.float32),
            pltpu.VMEM((EB,), jnp.int32),
            pltpu.VMEM((EB,), jnp.int32),
            pltpu.VMEM((2, 6, _L), jnp.float32),
            pltpu.VMEM((CAP,), jnp.int32),
            pltpu.VMEM((CAP,), jnp.float32),
            pltpu.VMEM((CAP,), jnp.int32),
            pltpu.VMEM((NSLOT, _L, D), jnp.float32),
            pltpu.VMEM((ZR, D), jnp.float32),
            pltpu.VMEM_SHARED((N, D), jnp.float32),
            pltpu.SemaphoreType.DMA,
            pltpu.SemaphoreType.DMA,
            pltpu.SemaphoreType.DMA,
            pltpu.SemaphoreType.DMA,
            pltpu.SemaphoreType.DMA,
            pltpu.SemaphoreType.DMA,
            pltpu.SemaphoreType.DMA,
            pltpu.SemaphoreType.DMA,
            pltpu.SemaphoreType.DMA,
        ],
    )
    def sc(y_hbm, src_hbm, dst_hbm, px_hbm, py_hbm, pz_hbm, kp_hbm, out_hbm,
           kpv, srcv, dstv, posb, idxb, inflb, dstb, rowb, zerov, outacc,
           psem, g0, g1, g2, g3, s0, s1, s2, s3):
        gsems = [g0, g1, g2, g3]
        ssems = [s0, s1, s2, s3]
        cid = lax.axis_index("c")
        sid = lax.axis_index("s")
        wid = cid * NS + sid
        ebase = wid * EPW

        pltpu.sync_copy(kp_hbm, kpv)

        def zb(i, _):
            for r in range(R8):
                zerov[i, pl.ds(r * _L, _L)] = jnp.zeros((_L,), jnp.float32)
            return 0
        lax.fori_loop(0, ZR, zb, 0)
        base = sid * RPT

        def zc(i, _):
            pltpu.sync_copy(zerov, outacc.at[pl.ds(base + i * ZR, ZR)])
            return 0
        lax.fori_loop(0, RPT // ZR, zc, 0)
        plsc.subcore_barrier()

        kxv = kpv[pl.ds(0, _L)]
        kyv = kpv[pl.ds(_L, _L)]
        kzv = kpv[pl.ds(2 * _L, _L)]
        kps = [(kxv[k], kyv[k], kzv[k]) for k in range(K)]

        def issue_pos(gl, b):
            sv = srcv[pl.ds(gl * _L, _L)]
            dv = dstv[pl.ds(gl * _L, _L)]
            pltpu.async_copy(px_hbm.at[sv], posb.at[b].at[0], psem)
            pltpu.async_copy(py_hbm.at[sv], posb.at[b].at[1], psem)
            pltpu.async_copy(pz_hbm.at[sv], posb.at[b].at[2], psem)
            pltpu.async_copy(px_hbm.at[dv], posb.at[b].at[3], psem)
            pltpu.async_copy(py_hbm.at[dv], posb.at[b].at[4], psem)
            pltpu.async_copy(pz_hbm.at[dv], posb.at[b].at[5], psem)

        def fire(i, tb):
            iv = idxb[pl.ds(i * _L, _L)]
            slot = (tb + i) % NSLOT
            for s in range(NSLOT):
                @pl.when(slot == s)
                def _():
                    @pl.when(tb + i >= NSLOT)
                    def _():
                        pltpu.make_async_copy(
                            y_hbm.at[pl.ds(0, _L)], rowb.at[s], ssems[s]).wait()
                    pltpu.async_copy(y_hbm.at[iv], rowb.at[s], gsems[s])

        def process(i, tb, inflv, dvec):
            slot = (tb + i) % NSLOT
            for s in range(NSLOT):
                @pl.when(slot == s)
                def _():
                    pltpu.make_async_copy(
                        y_hbm.at[pl.ds(0, _L)], rowb.at[s], gsems[s]).wait()
                    for j in range(_L):
                        f = inflv[j]
                        for r in range(R8):
                            rowb[s, j, pl.ds(r * _L, _L)] = (
                                rowb[s, j, pl.ds(r * _L, _L)] * f)
                    pltpu.async_copy(rowb.at[s], outacc.at[dvec],
                                     ssems[s], add=True)

        def body(g, carry):
            cnt, tb = carry
            gl = g % GPB
            b = g % 2

            @pl.when(gl == 0)
            def _():
                blk = g // GPB
                pltpu.sync_copy(src_hbm.at[pl.ds(ebase + blk * EB, EB)], srcv)
                pltpu.sync_copy(dst_hbm.at[pl.ds(ebase + blk * EB, EB)], dstv)
                issue_pos(0, b)

            sv = srcv[pl.ds(gl * _L, _L)]
            dv = dstv[pl.ds(gl * _L, _L)]

            for c in range(6):
                pltpu.make_async_copy(
                    px_hbm.at[pl.ds(0, _L)], posb.at[b].at[c], psem).wait()
            relx = posb[b, 0, :] - posb[b, 3, :]
            rely = posb[b, 1, :] - posb[b, 4, :]
            relz = posb[b, 2, :] - posb[b, 5, :]

            @pl.when(jnp.logical_and(g + 1 < GPW, gl + 1 < GPB))
            def _():
                issue_pos(gl + 1, 1 - b)

            for k in range(K):
                kx, ky, kz = kps[k]
                dx = relx - kx
                dy = rely - ky
                dz = relz - kz
                d2 = dx * dx + dy * dy + dz * dz + 1e-12
                dist = d2 * _rsqrt(d2)
                infl = jnp.maximum(0.0, 1.0 - dist * (1.0 / _SIGMA))
                m = infl > 0.0
                plsc.store_compressed(idxb.at[pl.ds(cnt, _L)], sv + k * N, mask=m)
                plsc.store_compressed(inflb.at[pl.ds(cnt, _L)], infl, mask=m)
                plsc.store_compressed(dstb.at[pl.ds(cnt, _L)], dv, mask=m)
                cnt = cnt + plsc.all_reduce_population_count(m)[0]

            nbat = cnt // _L

            def prime(i, _):
                fire(i, tb)
                return 0
            lax.fori_loop(0, jnp.minimum(nbat, NSLOT - 1), prime, 0)

            def bloop(i, _):
                @pl.when(i + (NSLOT - 1) < nbat)
                def _():
                    fire(i + (NSLOT - 1), tb)
                inflv = inflb[pl.ds(i * _L, _L)]
                dvec = dstb[pl.ds(i * _L, _L)]
                process(i, tb, inflv, dvec)
                return 0
            lax.fori_loop(0, nbat, bloop, 0)

            rem = cnt - nbat * _L
            iv = idxb[pl.ds(nbat * _L, _L)]
            fv = inflb[pl.ds(nbat * _L, _L)]
            dvv = dstb[pl.ds(nbat * _L, _L)]
            idxb[pl.ds(0, _L)] = iv
            inflb[pl.ds(0, _L)] = fv
            dstb[pl.ds(0, _L)] = dvv
            return rem, tb + nbat

        cnt, tb = lax.fori_loop(0, GPW, body, (jnp.int32(0), jnp.int32(0)))

        @pl.when(cnt > 0)
        def _():
            lane = lax.iota(jnp.int32, _L)
            m = lane < cnt
            iv = jnp.where(m, idxb[pl.ds(0, _L)], 0)
            fv = jnp.where(m, inflb[pl.ds(0, _L)], 0.0)
            dvv = jnp.where(m, dstb[pl.ds(0, _L)], 0)
            idxb[pl.ds(0, _L)] = iv
            inflb[pl.ds(0, _L)] = fv
            dstb[pl.ds(0, _L)] = dvv
            fire(0, tb)
            process(0, tb, fv, dvv)

        tbf = tb + jnp.where(cnt > 0, 1, 0)
        for s in range(NSLOT):
            @pl.when(tbf > s)
            def _():
                pltpu.make_async_copy(
                    y_hbm.at[pl.ds(0, _L)], rowb.at[s], ssems[s]).wait()
        plsc.subcore_barrier()

        pltpu.sync_copy(outacc.at[pl.ds(base, RPT)],
                        out_hbm.at[cid].at[pl.ds(base, RPT)])

    return sc


def kernel(x, pos, edge_index, kernel_points, weights):
    N, D_IN = x.shape
    K = kernel_points.shape[0]
    D_OUT = weights.shape[2]
    E = edge_index.shape[1]

    y = pl.pallas_call(
        _y_matmul_body,
        grid=(K,),
        in_specs=[
            pl.BlockSpec((N, D_IN), lambda k: (0, 0)),
            pl.BlockSpec((1, D_IN, D_OUT), lambda k: (k, 0, 0)),
        ],
        out_specs=pl.BlockSpec((1, N, D_OUT), lambda k: (k, 0, 0)),
        out_shape=jax.ShapeDtypeStruct((K, N, D_OUT), jnp.float32),
    )(x, weights)

    src = edge_index[0]
    dst = edge_index[1]
    kp = jnp.zeros((3, _L), jnp.float32).at[:, :K].set(kernel_points.T).reshape(3 * _L)

    sc = _make_sc_kernel(N, E, K, D_OUT)
    partials = sc(y.reshape(K * N, D_OUT), src, dst,
                  pos[:, 0], pos[:, 1], pos[:, 2], kp)
    return partials[0] + partials[1]


# speedup vs baseline: 18.1492x; 1.0198x over previous
import functools

import jax
import jax.numpy as jnp
from jax import lax
from jax.experimental import pallas as pl
from jax.experimental.pallas import tpu as pltpu
from jax.experimental.pallas import tpu_sc as plsc

_SIGMA = 0.5
_L = 16


def _y_matmul_body(x_ref, w_ref, o_ref):
    o_ref[0] = jnp.dot(x_ref[...], w_ref[0], preferred_element_type=jnp.float32)


def _rsqrt(d2):
    i = plsc.bitcast(d2, jnp.int32)
    i = jnp.int32(0x5F3759DF) - (i >> 1)
    y = plsc.bitcast(i, jnp.float32)
    for _ in range(2):
        y = y * (1.5 - 0.5 * d2 * y * y)
    return y


@functools.lru_cache(maxsize=None)
def _make_sc_kernel(N, E, K, D):
    NC, NS = 2, 16
    NW = NC * NS
    EPW = E // NW
    GPW = EPW // _L
    RPT = N // NS
    ZR = 25
    R8 = D // _L
    EB = 2000
    GPB = EB // _L
    CAP = 256
    NSLOT = 4

    mesh = plsc.VectorSubcoreMesh(core_axis_name="c", subcore_axis_name="s")

    @functools.partial(
        pl.kernel,
        out_type=jax.ShapeDtypeStruct((NC, N, D), jnp.float32),
        mesh=mesh,
        compiler_params=pltpu.CompilerParams(
            use_tc_tiling_on_sc=False, needs_layout_passes=False),
        scratch_types=[
            pltpu.VMEM((3 * _L,), jnp.float32),
            pltpu.VMEM((EB,), jnp.int32),
            pltpu.VMEM((EB,), jnp.int32),
            pltpu.VMEM((2, 6, _L), jnp.float32),
            pltpu.VMEM((2, CAP), jnp.int32),
            pltpu.VMEM((2, CAP), jnp.float32),
            pltpu.VMEM((2, CAP), jnp.int32),
            pltpu.VMEM((NSLOT, _L, D), jnp.float32),
            pltpu.VMEM((ZR, D), jnp.float32),
            pltpu.VMEM_SHARED((N, D), jnp.float32),
            pltpu.SemaphoreType.DMA,
            pltpu.SemaphoreType.DMA,
            pltpu.SemaphoreType.DMA,
            pltpu.SemaphoreType.DMA,
            pltpu.SemaphoreType.DMA,
            pltpu.SemaphoreType.DMA,
            pltpu.SemaphoreType.DMA,
            pltpu.SemaphoreType.DMA,
            pltpu.SemaphoreType.DMA,
        ],
    )
    def sc(y_hbm, src_hbm, dst_hbm, px_hbm, py_hbm, pz_hbm, kp_hbm, out_hbm,
           kpv, srcv, dstv, posb, idxb, inflb, dstb, rowb, zerov, outacc,
           psem, g0, g1, g2, g3, s0, s1, s2, s3):
        gsems = [g0, g1, g2, g3]
        ssems = [s0, s1, s2, s3]
        cid = lax.axis_index("c")
        sid = lax.axis_index("s")
        wid = cid * NS + sid
        ebase = wid * EPW

        pltpu.sync_copy(kp_hbm, kpv)

        def zb(i, _):
            for r in range(R8):
                zerov[i, pl.ds(r * _L, _L)] = jnp.zeros((_L,), jnp.float32)
            return 0
        lax.fori_loop(0, ZR, zb, 0)
        base = sid * RPT

        def zc(i, _):
            pltpu.sync_copy(zerov, outacc.at[pl.ds(base + i * ZR, ZR)])
            return 0
        lax.fori_loop(0, RPT // ZR, zc, 0)
        plsc.subcore_barrier()

        kxv = kpv[pl.ds(0, _L)]
        kyv = kpv[pl.ds(_L, _L)]
        kzv = kpv[pl.ds(2 * _L, _L)]
        kps = [(kxv[k], kyv[k], kzv[k]) for k in range(K)]

        def issue_pos(gl, b):
            sv = srcv[pl.ds(gl * _L, _L)]
            dv = dstv[pl.ds(gl * _L, _L)]
            pltpu.async_copy(px_hbm.at[sv], posb.at[b].at[0], psem)
            pltpu.async_copy(py_hbm.at[sv], posb.at[b].at[1], psem)
            pltpu.async_copy(pz_hbm.at[sv], posb.at[b].at[2], psem)
            pltpu.async_copy(px_hbm.at[dv], posb.at[b].at[3], psem)
            pltpu.async_copy(py_hbm.at[dv], posb.at[b].at[4], psem)
            pltpu.async_copy(pz_hbm.at[dv], posb.at[b].at[5], psem)

        def fire(pbuf, i, tb):
            iv = idxb[pbuf, pl.ds(i * _L, _L)]
            slot = (tb + i) % NSLOT
            for s in range(NSLOT):
                @pl.when(slot == s)
                def _():
                    @pl.when(tb + i >= NSLOT)
                    def _():
                        pltpu.make_async_copy(
                            y_hbm.at[pl.ds(0, _L)], rowb.at[s], ssems[s]).wait()
                    pltpu.async_copy(y_hbm.at[iv], rowb.at[s], gsems[s])

        def process(i, tb, inflv, dvec):
            slot = (tb + i) % NSLOT
            for s in range(NSLOT):
                @pl.when(slot == s)
                def _():
                    pltpu.make_async_copy(
                        y_hbm.at[pl.ds(0, _L)], rowb.at[s], gsems[s]).wait()
                    for j in range(_L):
                        f = inflv[j]
                        for r in range(R8):
                            rowb[s, j, pl.ds(r * _L, _L)] = (
                                rowb[s, j, pl.ds(r * _L, _L)] * f)
                    pltpu.async_copy(rowb.at[s], outacc.at[dvec],
                                     ssems[s], add=True)

        def run_batches(pbuf, pnb, tb):
            def bloop(i, _):
                @pl.when(i + (NSLOT - 1) < pnb)
                def _():
                    fire(pbuf, i + (NSLOT - 1), tb)
                inflv = inflb[pbuf, pl.ds(i * _L, _L)]
                dvec = dstb[pbuf, pl.ds(i * _L, _L)]
                process(i, tb, inflv, dvec)
                return 0
            lax.fori_loop(0, pnb, bloop, 0)

        def body(g, carry):
            rem, pnb, tb = carry
            gl = g % GPB
            b = g % 2
            pb = 1 - b

            run_batches(pb, pnb, tb)

            @pl.when(gl == 0)
            def _():
                blk = g // GPB
                pltpu.sync_copy(src_hbm.at[pl.ds(ebase + blk * EB, EB)], srcv)
                pltpu.sync_copy(dst_hbm.at[pl.ds(ebase + blk * EB, EB)], dstv)
                issue_pos(0, b)

            sv = srcv[pl.ds(gl * _L, _L)]
            dv = dstv[pl.ds(gl * _L, _L)]

            for c in range(6):
                pltpu.make_async_copy(
                    px_hbm.at[pl.ds(0, _L)], posb.at[b].at[c], psem).wait()
            relx = posb[b, 0, :] - posb[b, 3, :]
            rely = posb[b, 1, :] - posb[b, 4, :]
            relz = posb[b, 2, :] - posb[b, 5, :]

            @pl.when(jnp.logical_and(g + 1 < GPW, gl + 1 < GPB))
            def _():
                issue_pos(gl + 1, 1 - b)

            cnt = rem
            off = pnb * _L
            idxb[b, pl.ds(0, _L)] = idxb[pb, pl.ds(off, _L)]
            inflb[b, pl.ds(0, _L)] = inflb[pb, pl.ds(off, _L)]
            dstb[b, pl.ds(0, _L)] = dstb[pb, pl.ds(off, _L)]

            for k in range(K):
                kx, ky, kz = kps[k]
                dx = relx - kx
                dy = rely - ky
                dz = relz - kz
                d2 = dx * dx + dy * dy + dz * dz + 1e-12
                dist = d2 * _rsqrt(d2)
                infl = jnp.maximum(0.0, 1.0 - dist * (1.0 / _SIGMA))
                m = infl > 0.0
                plsc.store_compressed(
                    idxb.at[b].at[pl.ds(cnt, _L)], sv + k * N, mask=m)
                plsc.store_compressed(
                    inflb.at[b].at[pl.ds(cnt, _L)], infl, mask=m)
                plsc.store_compressed(
                    dstb.at[b].at[pl.ds(cnt, _L)], dv, mask=m)
                cnt = cnt + plsc.all_reduce_population_count(m)[0]

            nbat = cnt // _L
            tbn = tb + pnb

            def prime(i, _):
                fire(b, i, tbn)
                return 0
            lax.fori_loop(0, jnp.minimum(nbat, NSLOT - 1), prime, 0)

            return cnt - nbat * _L, nbat, tbn

        rem, pnb, tb = lax.fori_loop(
            0, GPW, body, (jnp.int32(0), jnp.int32(0), jnp.int32(0)))

        lb = 1 - (GPW % 2)
        run_batches(lb, pnb, tb)

        @pl.when(rem > 0)
        def _():
            off = pnb * _L
            lane = lax.iota(jnp.int32, _L)
            m = lane < rem
            iv = jnp.where(m, idxb[lb, pl.ds(off, _L)], 0)
            fv = jnp.where(m, inflb[lb, pl.ds(off, _L)], 0.0)
            dvv = jnp.where(m, dstb[lb, pl.ds(off, _L)], 0)
            idxb[lb, pl.ds(off, _L)] = iv
            inflb[lb, pl.ds(off, _L)] = fv
            dstb[lb, pl.ds(off, _L)] = dvv
            fire(lb, pnb, tb)
            process(pnb, tb, fv, dvv)

        tbf = tb + pnb + jnp.where(rem > 0, 1, 0)
        for s in range(NSLOT):
            @pl.when(tbf > s)
            def _():
                pltpu.make_async_copy(
                    y_hbm.at[pl.ds(0, _L)], rowb.at[s], ssems[s]).wait()
        plsc.subcore_barrier()

        pltpu.sync_copy(outacc.at[pl.ds(base, RPT)],
                        out_hbm.at[cid].at[pl.ds(base, RPT)])

    return sc


def kernel(x, pos, edge_index, kernel_points, weights):
    N, D_IN = x.shape
    K = kernel_points.shape[0]
    D_OUT = weights.shape[2]
    E = edge_index.shape[1]

    y = pl.pallas_call(
        _y_matmul_body,
        grid=(K,),
        in_specs=[
            pl.BlockSpec((N, D_IN), lambda k: (0, 0)),
            pl.BlockSpec((1, D_IN, D_OUT), lambda k: (k, 0, 0)),
        ],
        out_specs=pl.BlockSpec((1, N, D_OUT), lambda k: (k, 0, 0)),
        out_shape=jax.ShapeDtypeStruct((K, N, D_OUT), jnp.float32),
    )(x, weights)

    src = edge_index[0]
    dst = edge_index[1]
    kp = jnp.zeros((3, _L), jnp.float32).at[:, :K].set(kernel_points.T).reshape(3 * _L)

    sc = _make_sc_kernel(N, E, K, D_OUT)
    partials = sc(y.reshape(K * N, D_OUT), src, dst,
                  pos[:, 0], pos[:, 1], pos[:, 2], kp)
    return partials[0] + partials[1]
